# baseline (device time: 4951736 ns/iter reference)
import jax
import jax.numpy as jnp
from jax import lax
from jax.experimental import pallas as pl
from jax.experimental.pallas import tpu as pltpu

N_DEV = 4


def _ring_hop2(src_r, src_l):

    def body(sr_ref, sl_ref, or_ref, ol_ref, ss_r, rs_r, ss_l, rs_l):
        d = lax.axis_index("i")
        right = lax.rem(d + 1, N_DEV)
        left = lax.rem(d + N_DEV - 1, N_DEV)
        rdma_r = pltpu.make_async_remote_copy(
            src_ref=sr_ref, dst_ref=or_ref, send_sem=ss_r, recv_sem=rs_r,
            device_id=(right,), device_id_type=pl.DeviceIdType.MESH,
        )
        rdma_l = pltpu.make_async_remote_copy(
            src_ref=sl_ref, dst_ref=ol_ref, send_sem=ss_l, recv_sem=rs_l,
            device_id=(left,), device_id_type=pl.DeviceIdType.MESH,
        )
        rdma_r.start()
        rdma_l.start()
        rdma_r.wait()
        rdma_l.wait()

    return pl.pallas_call(
        body,
        out_shape=(
            jax.ShapeDtypeStruct(src_r.shape, src_r.dtype),
            jax.ShapeDtypeStruct(src_l.shape, src_l.dtype),
        ),
        in_specs=[
            pl.BlockSpec(memory_space=pl.ANY),
            pl.BlockSpec(memory_space=pl.ANY),
        ],
        out_specs=(
            pl.BlockSpec(memory_space=pl.ANY),
            pl.BlockSpec(memory_space=pl.ANY),
        ),
        scratch_shapes=[pltpu.SemaphoreType.DMA] * 4,
    )(src_r, src_l)


def _all_gather_into_out(own):
    m, n = own.shape
    h = m // 2

    def body(own_ref, out_ref, stage_a, stage_b, loc_sems, sa, ra, sb, rb):
        d = lax.axis_index("i")
        right = lax.rem(d + 1, N_DEV)
        left = lax.rem(d + N_DEV - 1, N_DEV)
        copies = []

        cp = pltpu.make_async_copy(
            own_ref, out_ref.at[pl.ds(d * m, m), :], loc_sems.at[0]
        )
        cp.start()
        copies.append(cp)

        src_a = own_ref.at[pl.ds(0, h), :]
        src_b = own_ref.at[pl.ds(h, h), :]
        for s in range(N_DEV - 1):
            rdma_a = pltpu.make_async_remote_copy(
                src_ref=src_a, dst_ref=stage_a.at[s],
                send_sem=sa.at[s], recv_sem=ra.at[s],
                device_id=(right,), device_id_type=pl.DeviceIdType.MESH,
            )
            rdma_b = pltpu.make_async_remote_copy(
                src_ref=src_b, dst_ref=stage_b.at[s],
                send_sem=sb.at[s], recv_sem=rb.at[s],
                device_id=(left,), device_id_type=pl.DeviceIdType.MESH,
            )
            rdma_a.start()
            rdma_b.start()
            rdma_a.wait()
            rdma_b.wait()
            ia = lax.rem(d + 2 * N_DEV - 1 - s, N_DEV)
            ib = lax.rem(d + 1 + s, N_DEV)
            cpa = pltpu.make_async_copy(
                stage_a.at[s],
                out_ref.at[pl.ds(ia * m, h), :],
                loc_sems.at[1 + 2 * s],
            )
            cpb = pltpu.make_async_copy(
                stage_b.at[s],
                out_ref.at[pl.ds(ib * m + h, h), :],
                loc_sems.at[2 + 2 * s],
            )
            cpa.start()
            cpb.start()
            copies += [cpa, cpb]
            src_a, src_b = stage_a.at[s], stage_b.at[s]

        for cp in copies:
            cp.wait()

    out, _, _ = pl.pallas_call(
        body,
        out_shape=(
            jax.ShapeDtypeStruct((N_DEV * m, n), own.dtype),
            jax.ShapeDtypeStruct((N_DEV - 1, h, n), own.dtype),
            jax.ShapeDtypeStruct((N_DEV - 1, h, n), own.dtype),
        ),
        in_specs=[pl.BlockSpec(memory_space=pl.ANY)],
        out_specs=(
            pl.BlockSpec(memory_space=pl.ANY),
            pl.BlockSpec(memory_space=pl.ANY),
            pl.BlockSpec(memory_space=pl.ANY),
        ),
        scratch_shapes=[
            pltpu.SemaphoreType.DMA((2 * N_DEV - 1,)),
            pltpu.SemaphoreType.DMA((N_DEV - 1,)),
            pltpu.SemaphoreType.DMA((N_DEV - 1,)),
            pltpu.SemaphoreType.DMA((N_DEV - 1,)),
            pltpu.SemaphoreType.DMA((N_DEV - 1,)),
        ],
    )(own)
    return out


def kernel(x, w_mat, scale_x, scale_w):
    d = lax.axis_index("i")

    partial = jnp.dot(
        x.astype(jnp.bfloat16),
        w_mat.astype(jnp.bfloat16),
        preferred_element_type=jnp.float32,
    )

    m_tot, n = partial.shape
    m = m_tot // N_DEV
    h = m // 2

    def upper(i):
        return lax.dynamic_slice_in_dim(
            partial, lax.rem(i, N_DEV) * m, h, axis=0
        )

    def lower(i):
        return lax.dynamic_slice_in_dim(
            partial, lax.rem(i, N_DEV) * m + h, h, axis=0
        )

    cur_a = upper(d + (N_DEV - 1))
    cur_b = lower(d + 1)
    for s in range(N_DEV - 1):
        ra, rb = _ring_hop2(cur_a, cur_b)
        cur_a = ra + upper(d + (2 * N_DEV - 2 - s))
        cur_b = rb + lower(d + 2 + s)

    scale = scale_x[0] * scale_w[0]

    def silu(acc):
        y = acc * scale
        return y * (1.0 / (1.0 + jnp.exp(-jnp.clip(y, -60.0, 60.0))))

    own = jnp.concatenate([silu(cur_a), silu(cur_b)], axis=0)

    return _all_gather_into_out(own)


# device time: 1395533 ns/iter; 3.5483x vs baseline; 3.5483x over previous
import jax
import jax.numpy as jnp
from jax import lax
from jax.experimental import pallas as pl
from jax.experimental.pallas import tpu as pltpu

N_DEV = 4


def _ring_hop2(src_r, src_l):

    def body(sr_ref, sl_ref, or_ref, ol_ref, ss_r, rs_r, ss_l, rs_l):
        d = lax.axis_index("i")
        right = lax.rem(d + 1, N_DEV)
        left = lax.rem(d + N_DEV - 1, N_DEV)
        rdma_r = pltpu.make_async_remote_copy(
            src_ref=sr_ref, dst_ref=or_ref, send_sem=ss_r, recv_sem=rs_r,
            device_id=(right,), device_id_type=pl.DeviceIdType.MESH,
        )
        rdma_l = pltpu.make_async_remote_copy(
            src_ref=sl_ref, dst_ref=ol_ref, send_sem=ss_l, recv_sem=rs_l,
            device_id=(left,), device_id_type=pl.DeviceIdType.MESH,
        )
        rdma_r.start()
        rdma_l.start()
        rdma_r.wait()
        rdma_l.wait()

    return pl.pallas_call(
        body,
        out_shape=(
            jax.ShapeDtypeStruct(src_r.shape, src_r.dtype),
            jax.ShapeDtypeStruct(src_l.shape, src_l.dtype),
        ),
        in_specs=[
            pl.BlockSpec(memory_space=pl.ANY),
            pl.BlockSpec(memory_space=pl.ANY),
        ],
        out_specs=(
            pl.BlockSpec(memory_space=pl.ANY),
            pl.BlockSpec(memory_space=pl.ANY),
        ),
        scratch_shapes=[pltpu.SemaphoreType.DMA] * 4,
    )(src_r, src_l)


def _merge_epilogue(didx, scale, recv_a, recv_b, partial, m, n):
    h = m // 2
    tile = 1024
    grid = (n // tile,)

    def body(didx_ref, scale_ref, ra_ref, rb_ref, p_ref, out_ref):
        s = scale_ref[0]

        def silu(acc):
            y = acc * s
            return y * (1.0 / (1.0 + jnp.exp(-jnp.clip(y, -60.0, 60.0))))

        out_ref[:h, :] = silu(ra_ref[...] + p_ref[:h, :])
        out_ref[h:, :] = silu(rb_ref[...] + p_ref[h:, :])

    grid_spec = pltpu.PrefetchScalarGridSpec(
        num_scalar_prefetch=2,
        grid=grid,
        in_specs=[
            pl.BlockSpec((h, tile), lambda j, dref, sref: (0, j)),
            pl.BlockSpec((h, tile), lambda j, dref, sref: (0, j)),
            pl.BlockSpec((m, tile), lambda j, dref, sref: (dref[0], j)),
        ],
        out_specs=pl.BlockSpec((m, tile), lambda j, dref, sref: (dref[0], j)),
    )
    return pl.pallas_call(
        body,
        grid_spec=grid_spec,
        out_shape=jax.ShapeDtypeStruct((N_DEV * m, n), jnp.float32),
    )(didx, scale, recv_a, recv_b, partial)


def _all_gather_inplace(buf, m):
    h = m // 2

    def body(in_ref, out_ref, sa, ra, sb, rb):
        del in_ref
        d = lax.axis_index("i")
        right = lax.rem(d + 1, N_DEV)
        left = lax.rem(d + N_DEV - 1, N_DEV)

        for s in range(N_DEV - 1):
            ia = lax.rem(d + N_DEV - s, N_DEV)
            ib = lax.rem(d + s, N_DEV)
            sl_a = (pl.ds(ia * m, h), slice(None))
            sl_b = (pl.ds(ib * m + h, h), slice(None))
            rdma_a = pltpu.make_async_remote_copy(
                src_ref=out_ref.at[sl_a], dst_ref=out_ref.at[sl_a],
                send_sem=sa.at[s], recv_sem=ra.at[s],
                device_id=(right,), device_id_type=pl.DeviceIdType.MESH,
            )
            rdma_b = pltpu.make_async_remote_copy(
                src_ref=out_ref.at[sl_b], dst_ref=out_ref.at[sl_b],
                send_sem=sb.at[s], recv_sem=rb.at[s],
                device_id=(left,), device_id_type=pl.DeviceIdType.MESH,
            )
            rdma_a.start()
            rdma_b.start()
            rdma_a.wait()
            rdma_b.wait()

    return pl.pallas_call(
        body,
        out_shape=jax.ShapeDtypeStruct(buf.shape, buf.dtype),
        in_specs=[pl.BlockSpec(memory_space=pl.ANY)],
        out_specs=pl.BlockSpec(memory_space=pl.ANY),
        input_output_aliases={0: 0},
        scratch_shapes=[pltpu.SemaphoreType.DMA((N_DEV - 1,))] * 4,
    )(buf)


def kernel(x, w_mat, scale_x, scale_w):
    d = lax.axis_index("i")

    partial = jnp.dot(
        x.astype(jnp.bfloat16),
        w_mat.astype(jnp.bfloat16),
        preferred_element_type=jnp.float32,
    )

    m_tot, n = partial.shape
    m = m_tot // N_DEV
    h = m // 2

    def upper(i):
        return lax.dynamic_slice_in_dim(
            partial, lax.rem(i, N_DEV) * m, h, axis=0
        )

    def lower(i):
        return lax.dynamic_slice_in_dim(
            partial, lax.rem(i, N_DEV) * m + h, h, axis=0
        )

    cur_a = upper(d + (N_DEV - 1))
    cur_b = lower(d + 1)
    for s in range(N_DEV - 2):
        ra, rb = _ring_hop2(cur_a, cur_b)
        cur_a = ra + upper(d + (2 * N_DEV - 2 - s))
        cur_b = rb + lower(d + 2 + s)
    ra, rb = _ring_hop2(cur_a, cur_b)

    scale = (scale_x * scale_w).astype(jnp.float32)
    out = _merge_epilogue(
        jnp.array([0], jnp.int32) + d, scale, ra, rb, partial, m, n
    )

    return _all_gather_inplace(out, m)
